# baseline (speedup 1.0000x reference)
import functools

import jax
import jax.numpy as jnp
from jax import lax
from jax.experimental import pallas as pl
from jax.experimental.pallas import tpu as pltpu
from jax.experimental.pallas import tpu_sc as plsc

N_NODES = 10000
N_EDGES = 320000
D = 128
BN_EPS = 1e-5

NC = 2
NS = 16
NW = NC * NS
CH = 128
STEPS = 80
IDXB = 20
NIB = STEPS // IDXB
E_PER_W = STEPS * CH
E_PAD = NW * E_PER_W
R_PAD = 10240
RPW = R_PAD // NS


def _sc_body(x_hbm, sidx_hbm, didx_hbm, zrow_hbm, zcnt_hbm,
             part_hbm, cnt_hbm,
             sA_v, dA_v, sB_v, dB_v, rows0_v, rows1_v, ones_v, acc_s, cnt_s,
             sem0, sem1, csem, isemA, isemB):
    cid = lax.axis_index("c")
    sid = lax.axis_index("s")
    wid = cid * NS + sid

    def idx_start(n, sbuf, dbuf, isem):
        pltpu.async_copy(sidx_hbm.at[wid, n], sbuf, isem)
        pltpu.async_copy(didx_hbm.at[wid, n], dbuf, isem)

    def idx_wait(n, sbuf, dbuf, isem):
        pltpu.make_async_copy(sidx_hbm.at[wid, n], sbuf, isem).wait()
        pltpu.make_async_copy(didx_hbm.at[wid, n], dbuf, isem).wait()

    idx_start(0, sA_v, dA_v, isemA)
    idx_start(1, sB_v, dB_v, isemB)

    pltpu.sync_copy(zrow_hbm, acc_s.at[pl.ds(sid * RPW, RPW)])
    pltpu.sync_copy(zcnt_hbm, cnt_s.at[pl.ds(sid * RPW, RPW)])

    for i in range(CH // 16):
        ones_v[pl.ds(i * 16, 16)] = jnp.full((16,), 1.0, jnp.float32)

    plsc.subcore_barrier()

    def gat_start(sbuf, j, buf, sem):
        pltpu.async_copy(x_hbm.at[sbuf.at[j]], buf, sem)

    def gat_wait(sbuf, j, buf, sem):
        pltpu.make_async_copy(x_hbm.at[sbuf.at[j]], buf, sem).wait()

    def scat(dbuf, j, buf):
        pltpu.async_copy(ones_v, cnt_s.at[dbuf.at[j]], csem, add=True)
        pltpu.sync_copy(buf, acc_s.at[dbuf.at[j]], add=True)
        pltpu.make_async_copy(ones_v, cnt_s.at[dbuf.at[j]], csem).wait()

    def process_block(n, sbuf, dbuf, isem):
        idx_wait(n, sbuf, dbuf, isem)
        gat_start(sbuf, 0, rows0_v, sem0)

        def pair(k, carry):
            j = 2 * k
            gat_start(sbuf, j + 1, rows1_v, sem1)
            gat_wait(sbuf, j, rows0_v, sem0)
            scat(dbuf, j, rows0_v)

            @pl.when(j + 2 < IDXB)
            def _():
                gat_start(sbuf, j + 2, rows0_v, sem0)

            gat_wait(sbuf, j + 1, rows1_v, sem1)
            scat(dbuf, j + 1, rows1_v)
            return carry

        lax.fori_loop(0, IDXB // 2, pair, 0)

        @pl.when(n + 2 < NIB)
        def _():
            idx_start(n + 2, sbuf, dbuf, isem)

    def superblock(p, carry):
        process_block(2 * p, sA_v, dA_v, isemA)
        process_block(2 * p + 1, sB_v, dB_v, isemB)
        return carry

    lax.fori_loop(0, NIB // 2, superblock, 0)

    plsc.subcore_barrier()

    pltpu.sync_copy(acc_s.at[pl.ds(sid * RPW, RPW)],
                    part_hbm.at[cid, pl.ds(sid * RPW, RPW)])
    pltpu.sync_copy(cnt_s.at[pl.ds(sid * RPW, RPW)],
                    cnt_hbm.at[cid, pl.ds(sid * RPW, RPW)])


_sc_agg = pl.kernel(
    _sc_body,
    out_type=[
        jax.ShapeDtypeStruct((NC, R_PAD, D), jnp.float32),
        jax.ShapeDtypeStruct((NC, R_PAD), jnp.float32),
    ],
    mesh=plsc.VectorSubcoreMesh(core_axis_name="c", subcore_axis_name="s"),
    scratch_types=[
        pltpu.VMEM((IDXB, CH), jnp.int32),
        pltpu.VMEM((IDXB, CH), jnp.int32),
        pltpu.VMEM((IDXB, CH), jnp.int32),
        pltpu.VMEM((IDXB, CH), jnp.int32),
        pltpu.VMEM((CH, D), jnp.float32),
        pltpu.VMEM((CH, D), jnp.float32),
        pltpu.VMEM((CH,), jnp.float32),
        pltpu.VMEM_SHARED((R_PAD, D), jnp.float32),
        pltpu.VMEM_SHARED((R_PAD,), jnp.float32),
        pltpu.SemaphoreType.DMA,
        pltpu.SemaphoreType.DMA,
        pltpu.SemaphoreType.DMA,
        pltpu.SemaphoreType.DMA,
        pltpu.SemaphoreType.DMA,
    ],
)

BM = 1000


def _tc_a_body(part_ref, cnt_ref, w_ref, b_ref, pre_ref, stat_ref):
    i = pl.program_id(0)
    c = cnt_ref[:, 0:1] + cnt_ref[:, 1:2]
    inv_ref = 1.0 / jnp.maximum(c, 1.0)
    has_ref = jnp.where(c > 0.0, 1.0, 0.0)
    agg = part_ref[0] + part_ref[1]
    scaled = agg * inv_ref
    pre = lax.dot_general(scaled, w_ref[...], (((1,), (1,)), ((), ())),
                          preferred_element_type=jnp.float32)
    pre = pre + has_ref * b_ref[...]
    pre_ref[...] = pre

    @pl.when(i == 0)
    def _():
        stat_ref[...] = jnp.zeros_like(stat_ref)

    stat_ref[0:1] += jnp.sum(pre, axis=0, keepdims=True)
    stat_ref[1:2] += jnp.sum(pre * pre, axis=0, keepdims=True)


_tc_a = pl.pallas_call(
    _tc_a_body,
    grid=(N_NODES // BM,),
    in_specs=[
        pl.BlockSpec((2, BM, D), lambda i: (0, i, 0)),
        pl.BlockSpec((BM, 2), lambda i: (i, 0)),
        pl.BlockSpec((D, D), lambda i: (0, 0)),
        pl.BlockSpec((1, D), lambda i: (0, 0)),
    ],
    out_specs=[
        pl.BlockSpec((BM, D), lambda i: (i, 0)),
        pl.BlockSpec((2, D), lambda i: (0, 0)),
    ],
    out_shape=[
        jax.ShapeDtypeStruct((N_NODES, D), jnp.float32),
        jax.ShapeDtypeStruct((2, D), jnp.float32),
    ],
)


def _tc_b_body(pre_ref, stat_ref, g_ref, bt_ref, out_ref):
    inv_n = 1.0 / N_NODES
    mu = stat_ref[0:1] * inv_n
    var = stat_ref[1:2] * inv_n - mu * mu
    scale = g_ref[...] * lax.rsqrt(var + BN_EPS)
    out_ref[...] = jnp.maximum((pre_ref[...] - mu) * scale + bt_ref[...], 0.0)


_tc_b = pl.pallas_call(
    _tc_b_body,
    grid=(N_NODES // BM,),
    in_specs=[
        pl.BlockSpec((BM, D), lambda i: (i, 0)),
        pl.BlockSpec((2, D), lambda i: (0, 0)),
        pl.BlockSpec((1, D), lambda i: (0, 0)),
        pl.BlockSpec((1, D), lambda i: (0, 0)),
    ],
    out_specs=pl.BlockSpec((BM, D), lambda i: (i, 0)),
    out_shape=jax.ShapeDtypeStruct((N_NODES, D), jnp.float32),
)


def kernel(x, edge_index, W, b, gamma, beta):
    ei = edge_index.astype(jnp.int32)
    pad = E_PAD - N_EDGES
    src = jnp.concatenate([ei[0], jnp.zeros((pad,), jnp.int32)])
    dst = jnp.concatenate([ei[1], jnp.full((pad,), R_PAD - 1, jnp.int32)])
    src = src.reshape(NW, NIB, IDXB, CH)
    dst = dst.reshape(NW, NIB, IDXB, CH)
    zrow = jnp.zeros((RPW, D), jnp.float32)
    zcnt = jnp.zeros((RPW,), jnp.float32)

    part, cnt = _sc_agg(x, src, dst, zrow, zcnt)

    pre, stat = _tc_a(part, cnt.T, W, b.reshape(1, D))
    return _tc_b(pre, stat, gamma.reshape(1, D), beta.reshape(1, D))


# speedup vs baseline: 2.0362x; 2.0362x over previous
import functools

import jax
import jax.numpy as jnp
from jax import lax
from jax.experimental import pallas as pl
from jax.experimental.pallas import tpu as pltpu
from jax.experimental.pallas import tpu_sc as plsc

N_NODES = 10000
N_EDGES = 320000
D = 128
DH = D // 2
BN_EPS = 1e-5

NC = 2
NS = 16
NW = NC * NS
CH = 128
STEPS = 80
IDXB = 20
NIB = STEPS // IDXB
E_PER_W = STEPS * CH
E_PAD = NW * E_PER_W
R_PAD = 10240
RPW = R_PAD // NS
XPW = 632


def _sc_body(x_hbm, sidx_hbm, didx_hbm, zrow_hbm, zcnt_hbm,
             part_hbm, cnt_hbm,
             sA_v, dA_v, sB_v, dB_v, rows0_v, rows1_v, ones_v,
             xs_s, acc_s, cnt_s,
             sem0, sem1, csem, isemA, isemB):
    cid = lax.axis_index("c")
    sid = lax.axis_index("s")
    wid = cid * NS + sid

    def idx_start(n, sbuf, dbuf, isem):
        pltpu.async_copy(sidx_hbm.at[wid, n], sbuf, isem)
        pltpu.async_copy(didx_hbm.at[wid, n], dbuf, isem)

    def idx_wait(n, sbuf, dbuf, isem):
        pltpu.make_async_copy(sidx_hbm.at[wid, n], sbuf, isem).wait()
        pltpu.make_async_copy(didx_hbm.at[wid, n], dbuf, isem).wait()

    for i in range(CH // 16):
        ones_v[pl.ds(i * 16, 16)] = jnp.full((16,), 1.0, jnp.float32)

    def gat_start(sbuf, j, buf, sem):
        pltpu.async_copy(xs_s.at[sbuf.at[j]], buf, sem)

    def gat_wait(sbuf, j, buf, sem):
        pltpu.make_async_copy(xs_s.at[sbuf.at[j]], buf, sem).wait()

    def scat(dbuf, j, buf, with_counts):
        if with_counts:
            pltpu.async_copy(ones_v, cnt_s.at[dbuf.at[j]], csem, add=True)
        pltpu.sync_copy(buf, acc_s.at[dbuf.at[j]], add=True)
        if with_counts:
            pltpu.make_async_copy(ones_v, cnt_s.at[dbuf.at[j]], csem).wait()

    def process_block(n, sbuf, dbuf, isem, with_counts):
        idx_wait(n, sbuf, dbuf, isem)
        gat_start(sbuf, 0, rows0_v, sem0)

        def pair(k, carry):
            j = 2 * k
            gat_start(sbuf, j + 1, rows1_v, sem1)
            gat_wait(sbuf, j, rows0_v, sem0)
            scat(dbuf, j, rows0_v, with_counts)

            @pl.when(j + 2 < IDXB)
            def _():
                gat_start(sbuf, j + 2, rows0_v, sem0)

            gat_wait(sbuf, j + 1, rows1_v, sem1)
            scat(dbuf, j + 1, rows1_v, with_counts)
            return carry

        lax.fori_loop(0, IDXB // 2, pair, 0)

        @pl.when(n + 2 < NIB)
        def _():
            idx_start(n + 2, sbuf, dbuf, isem)

    def run_pass(h, with_counts):
        @pl.when(sid < NS - 1)
        def _():
            pltpu.async_copy(x_hbm.at[pl.ds(sid * XPW, XPW), pl.ds(h * DH, DH)],
                             xs_s.at[pl.ds(sid * XPW, XPW)], sem0)

        @pl.when(sid == NS - 1)
        def _():
            pltpu.async_copy(
                x_hbm.at[pl.ds((NS - 1) * XPW, N_NODES - (NS - 1) * XPW),
                         pl.ds(h * DH, DH)],
                xs_s.at[pl.ds((NS - 1) * XPW, N_NODES - (NS - 1) * XPW)], sem0)

        idx_start(0, sA_v, dA_v, isemA)
        idx_start(1, sB_v, dB_v, isemB)
        pltpu.sync_copy(zrow_hbm, acc_s.at[pl.ds(sid * RPW, RPW)])
        if with_counts:
            pltpu.sync_copy(zcnt_hbm, cnt_s.at[pl.ds(sid * RPW, RPW)])

        @pl.when(sid < NS - 1)
        def _():
            pltpu.make_async_copy(
                x_hbm.at[pl.ds(sid * XPW, XPW), pl.ds(h * DH, DH)],
                xs_s.at[pl.ds(sid * XPW, XPW)], sem0).wait()

        @pl.when(sid == NS - 1)
        def _():
            pltpu.make_async_copy(
                x_hbm.at[pl.ds((NS - 1) * XPW, N_NODES - (NS - 1) * XPW),
                         pl.ds(h * DH, DH)],
                xs_s.at[pl.ds((NS - 1) * XPW, N_NODES - (NS - 1) * XPW)], sem0).wait()

        plsc.subcore_barrier()

        def superblock(p, carry):
            process_block(2 * p, sA_v, dA_v, isemA, with_counts)
            process_block(2 * p + 1, sB_v, dB_v, isemB, with_counts)
            return carry

        lax.fori_loop(0, NIB // 2, superblock, 0)
        plsc.subcore_barrier()

        pltpu.sync_copy(acc_s.at[pl.ds(sid * RPW, RPW)],
                        part_hbm.at[h, cid, pl.ds(sid * RPW, RPW)])
        if with_counts:
            pltpu.sync_copy(cnt_s.at[pl.ds(sid * RPW, RPW)],
                            cnt_hbm.at[cid, pl.ds(sid * RPW, RPW)])

    run_pass(0, True)
    plsc.subcore_barrier()
    run_pass(1, False)


_sc_agg = pl.kernel(
    _sc_body,
    out_type=[
        jax.ShapeDtypeStruct((2, NC, R_PAD, DH), jnp.float32),
        jax.ShapeDtypeStruct((NC, R_PAD), jnp.float32),
    ],
    mesh=plsc.VectorSubcoreMesh(core_axis_name="c", subcore_axis_name="s"),
    compiler_params=pltpu.CompilerParams(use_tc_tiling_on_sc=False),
    scratch_types=[
        pltpu.VMEM((IDXB, CH), jnp.int32),
        pltpu.VMEM((IDXB, CH), jnp.int32),
        pltpu.VMEM((IDXB, CH), jnp.int32),
        pltpu.VMEM((IDXB, CH), jnp.int32),
        pltpu.VMEM((CH, DH), jnp.float32),
        pltpu.VMEM((CH, DH), jnp.float32),
        pltpu.VMEM((CH,), jnp.float32),
        pltpu.VMEM_SHARED((N_NODES, DH), jnp.float32),
        pltpu.VMEM_SHARED((R_PAD, DH), jnp.float32),
        pltpu.VMEM_SHARED((R_PAD,), jnp.float32),
        pltpu.SemaphoreType.DMA,
        pltpu.SemaphoreType.DMA,
        pltpu.SemaphoreType.DMA,
        pltpu.SemaphoreType.DMA,
        pltpu.SemaphoreType.DMA,
    ],
)

BM = 1000


def _tc_a_body(part_ref, cnt_ref, w_ref, b_ref, pre_ref, stat_ref):
    i = pl.program_id(0)
    c = cnt_ref[:, 0:1] + cnt_ref[:, 1:2]
    inv = 1.0 / jnp.maximum(c, 1.0)
    has = jnp.where(c > 0.0, 1.0, 0.0)
    agg0 = (part_ref[0, 0] + part_ref[0, 1]) * inv
    agg1 = (part_ref[1, 0] + part_ref[1, 1]) * inv
    pre = lax.dot_general(agg0, w_ref[:, 0:DH], (((1,), (1,)), ((), ())),
                          preferred_element_type=jnp.float32)
    pre += lax.dot_general(agg1, w_ref[:, DH:D], (((1,), (1,)), ((), ())),
                           preferred_element_type=jnp.float32)
    pre = pre + has * b_ref[...]
    pre_ref[...] = pre

    @pl.when(i == 0)
    def _():
        stat_ref[...] = jnp.zeros_like(stat_ref)

    stat_ref[0:1] += jnp.sum(pre, axis=0, keepdims=True)
    stat_ref[1:2] += jnp.sum(pre * pre, axis=0, keepdims=True)


_tc_a = pl.pallas_call(
    _tc_a_body,
    grid=(N_NODES // BM,),
    in_specs=[
        pl.BlockSpec((2, NC, BM, DH), lambda i: (0, 0, i, 0)),
        pl.BlockSpec((BM, 2), lambda i: (i, 0)),
        pl.BlockSpec((D, D), lambda i: (0, 0)),
        pl.BlockSpec((1, D), lambda i: (0, 0)),
    ],
    out_specs=[
        pl.BlockSpec((BM, D), lambda i: (i, 0)),
        pl.BlockSpec((2, D), lambda i: (0, 0)),
    ],
    out_shape=[
        jax.ShapeDtypeStruct((N_NODES, D), jnp.float32),
        jax.ShapeDtypeStruct((2, D), jnp.float32),
    ],
)


def _tc_b_body(pre_ref, stat_ref, g_ref, bt_ref, out_ref):
    inv_n = 1.0 / N_NODES
    mu = stat_ref[0:1] * inv_n
    var = stat_ref[1:2] * inv_n - mu * mu
    scale = g_ref[...] * lax.rsqrt(var + BN_EPS)
    out_ref[...] = jnp.maximum((pre_ref[...] - mu) * scale + bt_ref[...], 0.0)


_tc_b = pl.pallas_call(
    _tc_b_body,
    grid=(N_NODES // BM,),
    in_specs=[
        pl.BlockSpec((BM, D), lambda i: (i, 0)),
        pl.BlockSpec((2, D), lambda i: (0, 0)),
        pl.BlockSpec((1, D), lambda i: (0, 0)),
        pl.BlockSpec((1, D), lambda i: (0, 0)),
    ],
    out_specs=pl.BlockSpec((BM, D), lambda i: (i, 0)),
    out_shape=jax.ShapeDtypeStruct((N_NODES, D), jnp.float32),
)


def kernel(x, edge_index, W, b, gamma, beta):
    ei = edge_index.astype(jnp.int32)
    pad = E_PAD - N_EDGES
    src = jnp.concatenate([ei[0], jnp.zeros((pad,), jnp.int32)])
    dst = jnp.concatenate([ei[1], jnp.full((pad,), R_PAD - 1, jnp.int32)])
    src = src.reshape(NW, NIB, IDXB, CH)
    dst = dst.reshape(NW, NIB, IDXB, CH)
    zrow = jnp.zeros((RPW, DH), jnp.float32)
    zcnt = jnp.zeros((RPW,), jnp.float32)

    part, cnt = _sc_agg(x, src, dst, zrow, zcnt)

    pre, stat = _tc_a(part, cnt.T, W, b.reshape(1, D))
    return _tc_b(pre, stat, gamma.reshape(1, D), beta.reshape(1, D))


# speedup vs baseline: 2.0844x; 1.0237x over previous
import functools

import jax
import jax.numpy as jnp
from jax import lax
from jax.experimental import pallas as pl
from jax.experimental.pallas import tpu as pltpu
from jax.experimental.pallas import tpu_sc as plsc

N_NODES = 10000
N_EDGES = 320000
D = 128
DH = D // 2
BN_EPS = 1e-5

NC = 2
NS = 16
NW = NC * NS
CH = 128
STEPS = 80
IDXB = 20
NIB = STEPS // IDXB
E_PER_W = STEPS * CH
E_PAD = NW * E_PER_W
R_PAD = 10240
RPW = R_PAD // NS
XPW = 632


def _sc_body(x_hbm, sidx_hbm, didx_hbm, zrow_hbm, zcnt_hbm,
             part_hbm, cnt_hbm,
             sA_v, dA_v, sB_v, dB_v, rows0_v, rows1_v, ones_v,
             xs_s, acc_s, cnt_s,
             sem0, sem1, csem, isemA, isemB):
    cid = lax.axis_index("c")
    sid = lax.axis_index("s")
    wid = cid * NS + sid

    def idx_start(n, sbuf, dbuf, isem):
        pltpu.async_copy(sidx_hbm.at[wid, n], sbuf, isem)
        pltpu.async_copy(didx_hbm.at[wid, n], dbuf, isem)

    def idx_wait(n, sbuf, dbuf, isem):
        pltpu.make_async_copy(sidx_hbm.at[wid, n], sbuf, isem).wait()
        pltpu.make_async_copy(didx_hbm.at[wid, n], dbuf, isem).wait()

    for i in range(CH // 16):
        ones_v[pl.ds(i * 16, 16)] = jnp.full((16,), 1.0, jnp.float32)

    def gat_start(sbuf, j, buf, sem):
        pltpu.async_copy(xs_s.at[sbuf.at[j]], buf, sem)

    def gat_wait(sbuf, j, buf, sem):
        pltpu.make_async_copy(xs_s.at[sbuf.at[j]], buf, sem).wait()

    def scat(dbuf, j, buf, with_counts):
        if with_counts:
            pltpu.async_copy(ones_v, cnt_s.at[dbuf.at[j]], csem, add=True)
        pltpu.sync_copy(buf, acc_s.at[dbuf.at[j]], add=True)
        if with_counts:
            pltpu.make_async_copy(ones_v, cnt_s.at[dbuf.at[j]], csem).wait()

    def process_block(n, sbuf, dbuf, isem, with_counts):
        idx_wait(n, sbuf, dbuf, isem)
        gat_start(sbuf, 0, rows0_v, sem0)

        def pair(k, carry):
            j = 2 * k
            gat_start(sbuf, j + 1, rows1_v, sem1)
            gat_wait(sbuf, j, rows0_v, sem0)
            scat(dbuf, j, rows0_v, with_counts)

            @pl.when(j + 2 < IDXB)
            def _():
                gat_start(sbuf, j + 2, rows0_v, sem0)

            gat_wait(sbuf, j + 1, rows1_v, sem1)
            scat(dbuf, j + 1, rows1_v, with_counts)
            return carry

        lax.fori_loop(0, IDXB // 2, pair, 0)

        @pl.when(n + 2 < NIB)
        def _():
            idx_start(n + 2, sbuf, dbuf, isem)

    def run_pass(h, with_counts):
        @pl.when(sid < NS - 1)
        def _():
            pltpu.async_copy(x_hbm.at[pl.ds(sid * XPW, XPW), pl.ds(h * DH, DH)],
                             xs_s.at[pl.ds(sid * XPW, XPW)], sem0)

        @pl.when(sid == NS - 1)
        def _():
            pltpu.async_copy(
                x_hbm.at[pl.ds((NS - 1) * XPW, N_NODES - (NS - 1) * XPW),
                         pl.ds(h * DH, DH)],
                xs_s.at[pl.ds((NS - 1) * XPW, N_NODES - (NS - 1) * XPW)], sem0)

        idx_start(0, sA_v, dA_v, isemA)
        idx_start(1, sB_v, dB_v, isemB)
        pltpu.sync_copy(zrow_hbm, acc_s.at[pl.ds(sid * RPW, RPW)])
        if with_counts:
            pltpu.sync_copy(zcnt_hbm, cnt_s.at[pl.ds(sid * RPW, RPW)])

        @pl.when(sid < NS - 1)
        def _():
            pltpu.make_async_copy(
                x_hbm.at[pl.ds(sid * XPW, XPW), pl.ds(h * DH, DH)],
                xs_s.at[pl.ds(sid * XPW, XPW)], sem0).wait()

        @pl.when(sid == NS - 1)
        def _():
            pltpu.make_async_copy(
                x_hbm.at[pl.ds((NS - 1) * XPW, N_NODES - (NS - 1) * XPW),
                         pl.ds(h * DH, DH)],
                xs_s.at[pl.ds((NS - 1) * XPW, N_NODES - (NS - 1) * XPW)], sem0).wait()

        plsc.subcore_barrier()

        def superblock(p, carry):
            process_block(2 * p, sA_v, dA_v, isemA, with_counts)
            process_block(2 * p + 1, sB_v, dB_v, isemB, with_counts)
            return carry

        lax.fori_loop(0, NIB // 2, superblock, 0)
        plsc.subcore_barrier()

        pltpu.sync_copy(acc_s.at[pl.ds(sid * RPW, RPW)],
                        part_hbm.at[h, cid, pl.ds(sid * RPW, RPW)])
        if with_counts:
            pltpu.sync_copy(cnt_s.at[pl.ds(sid * RPW, RPW)],
                            cnt_hbm.at[cid, pl.ds(sid * RPW, RPW)])

    run_pass(0, True)
    plsc.subcore_barrier()
    run_pass(1, False)


_sc_agg = pl.kernel(
    _sc_body,
    out_type=[
        jax.ShapeDtypeStruct((2, NC, R_PAD, DH), jnp.float32),
        jax.ShapeDtypeStruct((NC, R_PAD), jnp.float32),
    ],
    mesh=plsc.VectorSubcoreMesh(core_axis_name="c", subcore_axis_name="s"),
    compiler_params=pltpu.CompilerParams(use_tc_tiling_on_sc=False),
    scratch_types=[
        pltpu.VMEM((IDXB, CH), jnp.int32),
        pltpu.VMEM((IDXB, CH), jnp.int32),
        pltpu.VMEM((IDXB, CH), jnp.int32),
        pltpu.VMEM((IDXB, CH), jnp.int32),
        pltpu.VMEM((CH, DH), jnp.float32),
        pltpu.VMEM((CH, DH), jnp.float32),
        pltpu.VMEM((CH,), jnp.float32),
        pltpu.VMEM_SHARED((N_NODES, DH), jnp.float32),
        pltpu.VMEM_SHARED((R_PAD, DH), jnp.float32),
        pltpu.VMEM_SHARED((R_PAD,), jnp.float32),
        pltpu.SemaphoreType.DMA,
        pltpu.SemaphoreType.DMA,
        pltpu.SemaphoreType.DMA,
        pltpu.SemaphoreType.DMA,
        pltpu.SemaphoreType.DMA,
    ],
)

BM = 1000
NB = N_NODES // BM


def _tc_body(part_ref, cnt_ref, w_ref, b_ref, g_ref, bt_ref, out_ref,
             pre_vmem, stat_vmem):
    i = pl.program_id(0)

    @pl.when(i < NB)
    def _():
        c = cnt_ref[:, 0:1] + cnt_ref[:, 1:2]
        inv = 1.0 / jnp.maximum(c, 1.0)
        has = jnp.where(c > 0.0, 1.0, 0.0)
        agg0 = (part_ref[0, 0] + part_ref[0, 1]) * inv
        agg1 = (part_ref[1, 0] + part_ref[1, 1]) * inv
        pre = lax.dot_general(agg0, w_ref[:, 0:DH], (((1,), (1,)), ((), ())),
                              preferred_element_type=jnp.float32)
        pre += lax.dot_general(agg1, w_ref[:, DH:D], (((1,), (1,)), ((), ())),
                               preferred_element_type=jnp.float32)
        pre = pre + has * b_ref[...]
        pre_vmem[pl.ds(i * BM, BM), :] = pre

        @pl.when(i == 0)
        def _():
            stat_vmem[...] = jnp.zeros_like(stat_vmem)

        stat_vmem[0:1] += jnp.sum(pre, axis=0, keepdims=True)
        stat_vmem[1:2] += jnp.sum(pre * pre, axis=0, keepdims=True)

    @pl.when(i >= NB)
    def _():
        inv_n = 1.0 / N_NODES
        mu = stat_vmem[0:1] * inv_n
        var = stat_vmem[1:2] * inv_n - mu * mu
        scale = g_ref[...] * lax.rsqrt(var + BN_EPS)
        pre = pre_vmem[pl.ds((i - NB) * BM, BM), :]
        out_ref[...] = jnp.maximum((pre - mu) * scale + bt_ref[...], 0.0)


_tc_fused = pl.pallas_call(
    _tc_body,
    grid=(2 * NB,),
    in_specs=[
        pl.BlockSpec((2, NC, BM, DH), lambda i: (0, 0, jnp.minimum(i, NB - 1), 0)),
        pl.BlockSpec((BM, 2), lambda i: (jnp.minimum(i, NB - 1), 0)),
        pl.BlockSpec((D, D), lambda i: (0, 0)),
        pl.BlockSpec((1, D), lambda i: (0, 0)),
        pl.BlockSpec((1, D), lambda i: (0, 0)),
        pl.BlockSpec((1, D), lambda i: (0, 0)),
    ],
    out_specs=pl.BlockSpec((BM, D), lambda i: (jnp.maximum(i - NB, 0), 0)),
    out_shape=jax.ShapeDtypeStruct((N_NODES, D), jnp.float32),
    scratch_shapes=[
        pltpu.VMEM((N_NODES, D), jnp.float32),
        pltpu.VMEM((2, D), jnp.float32),
    ],
)


def kernel(x, edge_index, W, b, gamma, beta):
    ei = edge_index.astype(jnp.int32)
    pad = E_PAD - N_EDGES
    src = jnp.concatenate([ei[0], jnp.zeros((pad,), jnp.int32)])
    dst = jnp.concatenate([ei[1], jnp.full((pad,), R_PAD - 1, jnp.int32)])
    src = src.reshape(NW, NIB, IDXB, CH)
    dst = dst.reshape(NW, NIB, IDXB, CH)
    zrow = jnp.zeros((RPW, DH), jnp.float32)
    zcnt = jnp.zeros((RPW,), jnp.float32)

    part, cnt = _sc_agg(x, src, dst, zrow, zcnt)

    return _tc_fused(part, cnt.T, W, b.reshape(1, D),
                     gamma.reshape(1, D), beta.reshape(1, D))


# speedup vs baseline: 2.2630x; 1.0857x over previous
import functools

import jax
import jax.numpy as jnp
from jax import lax
from jax.experimental import pallas as pl
from jax.experimental.pallas import tpu as pltpu
from jax.experimental.pallas import tpu_sc as plsc

N_NODES = 10000
N_EDGES = 320000
D = 128
DH = D // 2
BN_EPS = 1e-5

NC = 2
NS = 16
NW = NC * NS
CH = 128
STEPS = 80
IDXB = 16
NIB = STEPS // IDXB
E_PER_W = STEPS * CH
E_PAD = NW * E_PER_W
R_PAD = 10240
RPW = R_PAD // NS
XPW = 632


def _sc_body(x_hbm, sidx_hbm, didx_hbm, zrow_hbm, zcnt_hbm,
             part_hbm, cnt_hbm,
             sA_v, dA_v, sB_v, dB_v, rows0_v, rows1_v, rows2_v, rows3_v,
             ones_v, xs_s, acc_s, cnt_s,
             g0, g1, g2, g3, s0, s1, s2, s3, csem, isemA, isemB):
    cid = lax.axis_index("c")
    sid = lax.axis_index("s")
    wid = cid * NS + sid
    rows = (rows0_v, rows1_v, rows2_v, rows3_v)
    gsem = (g0, g1, g2, g3)
    ssem = (s0, s1, s2, s3)

    def idx_start(n, sbuf, dbuf, isem):
        pltpu.async_copy(sidx_hbm.at[wid, n], sbuf, isem)
        pltpu.async_copy(didx_hbm.at[wid, n], dbuf, isem)

    def idx_wait(n, sbuf, dbuf, isem):
        pltpu.make_async_copy(sidx_hbm.at[wid, n], sbuf, isem).wait()
        pltpu.make_async_copy(didx_hbm.at[wid, n], dbuf, isem).wait()

    for i in range(CH // 16):
        ones_v[pl.ds(i * 16, 16)] = jnp.full((16,), 1.0, jnp.float32)

    def gat_start(sbuf, j, buf, sem):
        pltpu.async_copy(xs_s.at[sbuf.at[j]], buf, sem)

    def gat_wait(sbuf, j, buf, sem):
        pltpu.make_async_copy(xs_s.at[sbuf.at[j]], buf, sem).wait()

    def scat_start(dbuf, j, buf, sem, with_counts):
        pltpu.async_copy(buf, acc_s.at[dbuf.at[j]], sem, add=True)
        if with_counts:
            pltpu.async_copy(ones_v, cnt_s.at[dbuf.at[j]], csem, add=True)

    def scat_wait(dbuf, j, buf, sem):
        pltpu.make_async_copy(buf, acc_s.at[dbuf.at[j]], sem).wait()

    def process_block(n, sbuf, dbuf, isem, with_counts):
        idx_wait(n, sbuf, dbuf, isem)
        gat_start(sbuf, 0, rows[0], gsem[0])

        def quad(k, carry):
            for m in range(4):
                i = 4 * k + m

                @pl.when(i >= 3)
                def _(i=i, m=m):
                    scat_wait(dbuf, i - 3, rows[(m + 1) % 4], ssem[(m + 1) % 4])

                @pl.when(i + 1 < IDXB)
                def _(i=i, m=m):
                    gat_start(sbuf, i + 1, rows[(m + 1) % 4], gsem[(m + 1) % 4])

                gat_wait(sbuf, i, rows[m], gsem[m])
                scat_start(dbuf, i, rows[m], ssem[m], with_counts)
            return carry

        lax.fori_loop(0, IDXB // 4, quad, 0)

        for j in range(IDXB - 3, IDXB):
            scat_wait(dbuf, j, rows[j % 4], ssem[j % 4])
        if with_counts:
            def cdrain(_, carry):
                pltpu.make_async_copy(ones_v, cnt_s.at[dbuf.at[0]], csem).wait()
                return carry
            lax.fori_loop(0, IDXB, cdrain, 0)

        if n + 2 < NIB:
            idx_start(n + 2, sbuf, dbuf, isem)

    def run_pass(h, with_counts):
        @pl.when(sid < NS - 1)
        def _():
            pltpu.async_copy(x_hbm.at[pl.ds(sid * XPW, XPW), pl.ds(h * DH, DH)],
                             xs_s.at[pl.ds(sid * XPW, XPW)], g0)

        @pl.when(sid == NS - 1)
        def _():
            pltpu.async_copy(
                x_hbm.at[pl.ds((NS - 1) * XPW, N_NODES - (NS - 1) * XPW),
                         pl.ds(h * DH, DH)],
                xs_s.at[pl.ds((NS - 1) * XPW, N_NODES - (NS - 1) * XPW)], g0)

        idx_start(0, sA_v, dA_v, isemA)
        idx_start(1, sB_v, dB_v, isemB)
        pltpu.sync_copy(zrow_hbm, acc_s.at[pl.ds(sid * RPW, RPW)])
        if with_counts:
            pltpu.sync_copy(zcnt_hbm, cnt_s.at[pl.ds(sid * RPW, RPW)])

        @pl.when(sid < NS - 1)
        def _():
            pltpu.make_async_copy(
                x_hbm.at[pl.ds(sid * XPW, XPW), pl.ds(h * DH, DH)],
                xs_s.at[pl.ds(sid * XPW, XPW)], g0).wait()

        @pl.when(sid == NS - 1)
        def _():
            pltpu.make_async_copy(
                x_hbm.at[pl.ds((NS - 1) * XPW, N_NODES - (NS - 1) * XPW),
                         pl.ds(h * DH, DH)],
                xs_s.at[pl.ds((NS - 1) * XPW, N_NODES - (NS - 1) * XPW)], g0).wait()

        plsc.subcore_barrier()

        for n in range(NIB):
            if n % 2 == 0:
                process_block(n, sA_v, dA_v, isemA, with_counts)
            else:
                process_block(n, sB_v, dB_v, isemB, with_counts)
        plsc.subcore_barrier()

        pltpu.sync_copy(acc_s.at[pl.ds(sid * RPW, RPW)],
                        part_hbm.at[h, cid, pl.ds(sid * RPW, RPW)])
        if with_counts:
            pltpu.sync_copy(cnt_s.at[pl.ds(sid * RPW, RPW)],
                            cnt_hbm.at[cid, pl.ds(sid * RPW, RPW)])

    run_pass(0, True)
    plsc.subcore_barrier()
    run_pass(1, False)


_sc_agg = pl.kernel(
    _sc_body,
    out_type=[
        jax.ShapeDtypeStruct((2, NC, R_PAD, DH), jnp.float32),
        jax.ShapeDtypeStruct((NC, R_PAD), jnp.float32),
    ],
    mesh=plsc.VectorSubcoreMesh(core_axis_name="c", subcore_axis_name="s"),
    compiler_params=pltpu.CompilerParams(use_tc_tiling_on_sc=False),
    scratch_types=[
        pltpu.VMEM((IDXB, CH), jnp.int32),
        pltpu.VMEM((IDXB, CH), jnp.int32),
        pltpu.VMEM((IDXB, CH), jnp.int32),
        pltpu.VMEM((IDXB, CH), jnp.int32),
        pltpu.VMEM((CH, DH), jnp.float32),
        pltpu.VMEM((CH, DH), jnp.float32),
        pltpu.VMEM((CH, DH), jnp.float32),
        pltpu.VMEM((CH, DH), jnp.float32),
        pltpu.VMEM((CH,), jnp.float32),
        pltpu.VMEM_SHARED((N_NODES, DH), jnp.float32),
        pltpu.VMEM_SHARED((R_PAD, DH), jnp.float32),
        pltpu.VMEM_SHARED((R_PAD,), jnp.float32),
        pltpu.SemaphoreType.DMA,
        pltpu.SemaphoreType.DMA,
        pltpu.SemaphoreType.DMA,
        pltpu.SemaphoreType.DMA,
        pltpu.SemaphoreType.DMA,
        pltpu.SemaphoreType.DMA,
        pltpu.SemaphoreType.DMA,
        pltpu.SemaphoreType.DMA,
        pltpu.SemaphoreType.DMA,
        pltpu.SemaphoreType.DMA,
        pltpu.SemaphoreType.DMA,
    ],
)

BM = 1000
NB = N_NODES // BM


def _tc_body(part_ref, cnt_ref, w_ref, b_ref, g_ref, bt_ref, out_ref,
             pre_vmem, stat_vmem):
    i = pl.program_id(0)

    @pl.when(i < NB)
    def _():
        c = cnt_ref[:, 0:1] + cnt_ref[:, 1:2]
        inv = 1.0 / jnp.maximum(c, 1.0)
        has = jnp.where(c > 0.0, 1.0, 0.0)
        agg0 = (part_ref[0, 0] + part_ref[0, 1]) * inv
        agg1 = (part_ref[1, 0] + part_ref[1, 1]) * inv
        pre = lax.dot_general(agg0, w_ref[:, 0:DH], (((1,), (1,)), ((), ())),
                              preferred_element_type=jnp.float32)
        pre += lax.dot_general(agg1, w_ref[:, DH:D], (((1,), (1,)), ((), ())),
                               preferred_element_type=jnp.float32)
        pre = pre + has * b_ref[...]
        pre_vmem[pl.ds(i * BM, BM), :] = pre

        @pl.when(i == 0)
        def _():
            stat_vmem[...] = jnp.zeros_like(stat_vmem)

        stat_vmem[0:1] += jnp.sum(pre, axis=0, keepdims=True)
        stat_vmem[1:2] += jnp.sum(pre * pre, axis=0, keepdims=True)

    @pl.when(i >= NB)
    def _():
        inv_n = 1.0 / N_NODES
        mu = stat_vmem[0:1] * inv_n
        var = stat_vmem[1:2] * inv_n - mu * mu
        scale = g_ref[...] * lax.rsqrt(var + BN_EPS)
        pre = pre_vmem[pl.ds((i - NB) * BM, BM), :]
        out_ref[...] = jnp.maximum((pre - mu) * scale + bt_ref[...], 0.0)


_tc_fused = pl.pallas_call(
    _tc_body,
    grid=(2 * NB,),
    in_specs=[
        pl.BlockSpec((2, NC, BM, DH), lambda i: (0, 0, jnp.minimum(i, NB - 1), 0)),
        pl.BlockSpec((BM, 2), lambda i: (jnp.minimum(i, NB - 1), 0)),
        pl.BlockSpec((D, D), lambda i: (0, 0)),
        pl.BlockSpec((1, D), lambda i: (0, 0)),
        pl.BlockSpec((1, D), lambda i: (0, 0)),
        pl.BlockSpec((1, D), lambda i: (0, 0)),
    ],
    out_specs=pl.BlockSpec((BM, D), lambda i: (jnp.maximum(i - NB, 0), 0)),
    out_shape=jax.ShapeDtypeStruct((N_NODES, D), jnp.float32),
    scratch_shapes=[
        pltpu.VMEM((N_NODES, D), jnp.float32),
        pltpu.VMEM((2, D), jnp.float32),
    ],
)


def kernel(x, edge_index, W, b, gamma, beta):
    ei = edge_index.astype(jnp.int32)
    pad = E_PAD - N_EDGES
    src = jnp.concatenate([ei[0], jnp.zeros((pad,), jnp.int32)])
    dst = jnp.concatenate([ei[1], jnp.full((pad,), R_PAD - 1, jnp.int32)])
    src = src.reshape(NW, NIB, IDXB, CH)
    dst = dst.reshape(NW, NIB, IDXB, CH)
    zrow = jnp.zeros((RPW, DH), jnp.float32)
    zcnt = jnp.zeros((RPW,), jnp.float32)

    part, cnt = _sc_agg(x, src, dst, zrow, zcnt)

    return _tc_fused(part, cnt.T, W, b.reshape(1, D),
                     gamma.reshape(1, D), beta.reshape(1, D))
